# baseline (speedup 1.0000x reference)
import functools
import numpy as np
import jax
import jax.numpy as jnp
from jax import lax
from jax.experimental import pallas as pl
from jax.experimental.pallas import tpu as pltpu

_C1 = 96
_CE = 96
_CAND = _C1 * (_C1 - 1) // 2
_RPAD = 8
_CPAD = 576
_NEG = -1e30
_G = 8


def _body(ia_ref, ja_ref, w_ref, b_ref, lp_ref, gp_ref, x_ref, out_ref,
          sel_ref, *, hw):
    bidx = pl.program_id(0)
    og = pl.program_id(1)

    @pl.when(jnp.logical_and(bidx == 0, og == 0))
    def _():
        z = lp_ref[...] + gp_ref[...]
        zmax = jnp.max(z)
        zmin = jnp.min(jnp.where(z < -1e29, zmax, z))

        def bis(_, carry):
            lo, hi = carry
            mid = 0.5 * (lo + hi)
            cnt = jnp.sum((z >= mid).astype(jnp.float32))
            take = cnt >= float(_CE)
            return (jnp.where(take, mid, lo), jnp.where(take, hi, mid))

        thr, _hi = lax.fori_loop(0, 64, bis, (zmin - 1.0, zmax + 1.0))
        selmask = z >= thr
        idx2d = (lax.broadcasted_iota(jnp.int32, (_RPAD, _CPAD), 0) * _CPAD
                 + lax.broadcasted_iota(jnp.int32, (_RPAD, _CPAD), 1))

        def ext(e, prev):
            nxt = jnp.min(jnp.where(jnp.logical_and(selmask, idx2d > prev),
                                    idx2d, jnp.int32(1 << 30)))
            sel_ref[0, e] = ia_ref[nxt]
            sel_ref[1, e] = ja_ref[nxt]
            return nxt

        lax.fori_loop(0, _CE, ext, jnp.int32(-1))

    inv = 1.0 / float(hw)

    def write(g, v):
        m = jnp.sum(v) * inv
        ex2 = jnp.sum(v * v) * inv
        var = ex2 - m * m
        o = og * _G + g
        scale = w_ref[o] * lax.rsqrt(var + 1e-5)
        out_ref[0, g] = v * scale + (b_ref[o] - m * scale)

    @pl.when(og < _C1 // _G)
    def _():
        for g in range(_G):
            write(g, x_ref[0, og * _G + g])

    @pl.when(og >= _C1 // _G)
    def _():
        for g in range(_G):
            e = og * _G + g - _C1
            write(g, x_ref[0, sel_ref[0, e]] * x_ref[0, sel_ref[1, e]])


@jax.jit
def kernel(x, logits, tau, in_weight, in_bias):
    B, C1, H, W = x.shape
    HW = H * W

    gumbels = -jnp.log(
        jax.random.exponential(jax.random.key(42), (_CAND,), dtype=jnp.float32))
    i_np, j_np = np.triu_indices(_C1, k=1)

    def padi(v):
        out = np.zeros((_RPAD * _CPAD,), dtype=np.int32)
        out[: v.shape[0]] = v
        return out

    lp = jnp.zeros((_RPAD * _CPAD,), jnp.float32).at[:_CAND].set(logits)
    lp = lp.reshape(_RPAD, _CPAD)
    gp = jnp.asarray(
        np.full((_RPAD * _CPAD,), _NEG, dtype=np.float32)
    ).at[:_CAND].set(gumbels).reshape(_RPAD, _CPAD)
    ia = jnp.asarray(padi(i_np.astype(np.int32)))
    ja = jnp.asarray(padi(j_np.astype(np.int32)))

    grid_spec = pltpu.PrefetchScalarGridSpec(
        num_scalar_prefetch=4,
        grid=(B, (_C1 + _CE) // _G),
        in_specs=[
            pl.BlockSpec((_RPAD, _CPAD), lambda b, o, *_: (0, 0)),
            pl.BlockSpec((_RPAD, _CPAD), lambda b, o, *_: (0, 0)),
            pl.BlockSpec((1, C1, H, W), lambda b, o, *_: (b, 0, 0, 0)),
        ],
        out_specs=pl.BlockSpec((1, _G, H, W), lambda b, o, *_: (b, o, 0, 0)),
        scratch_shapes=[pltpu.SMEM((2, 128), jnp.int32)],
    )
    return pl.pallas_call(
        functools.partial(_body, hw=HW),
        grid_spec=grid_spec,
        out_shape=jax.ShapeDtypeStruct((B, _C1 + _CE, H, W), jnp.float32),
    )(ia, ja, in_weight, in_bias, lp, gp, x)


# speedup vs baseline: 1.1083x; 1.1083x over previous
import functools
import numpy as np
import jax
import jax.numpy as jnp
from jax import lax
from jax.experimental import pallas as pl
from jax.experimental.pallas import tpu as pltpu

_C1 = 96
_CE = 96
_CAND = _C1 * (_C1 - 1) // 2
_RPAD = 8
_CPAD = 576
_NEG = -1e30
_G = 8


def _sel_body(lp_ref, gp_ref, ia_ref, ja_ref, u_ref, out_ref):
    z = lp_ref[...] + gp_ref[...]
    zmax = jnp.max(z)
    zreal = jnp.where(z < -1e29, zmax, z)
    zmin = jnp.min(zreal)

    def bis(_, carry):
        lo, hi = carry
        mid = 0.5 * (lo + hi)
        cnt = jnp.sum((z >= mid).astype(jnp.float32))
        take = cnt >= float(_CE)
        return (jnp.where(take, mid, lo), jnp.where(take, hi, mid))

    lo, _ = lax.fori_loop(0, 64, bis, (zmin - 1.0, zmax + 1.0))
    mask = (z >= lo).astype(jnp.float32)

    within = jnp.dot(mask, u_ref[...], preferred_element_type=jnp.float32)
    rowsum = jnp.sum(mask, axis=1)
    r = lax.broadcasted_iota(jnp.int32, (_RPAD, _RPAD), 0)
    rp = lax.broadcasted_iota(jnp.int32, (_RPAD, _RPAD), 1)
    offs = jnp.sum(jnp.where(rp < r, rowsum[None, :], 0.0), axis=1)
    ranks = (within + offs[:, None]) * mask
    ranks_i = ranks.astype(jnp.int32)

    e = lax.broadcasted_iota(jnp.int32, (_RPAD, _CPAD, 128), 2) + 1
    onehot = (ranks_i[:, :, None] == e).astype(jnp.float32)
    isel = jnp.sum(jnp.sum(ia_ref[...][:, :, None] * onehot, axis=1), axis=0,
                   keepdims=True)
    jsel = jnp.sum(jnp.sum(ja_ref[...][:, :, None] * onehot, axis=1), axis=0,
                   keepdims=True)
    out_ref[0:1, :] = isel.astype(jnp.int32)
    out_ref[1:2, :] = jsel.astype(jnp.int32)


def _norm_body(sel1_ref, sel2_ref, w_ref, b_ref, x_ref, out_ref, *, hw):
    og = pl.program_id(1)
    inv = 1.0 / float(hw)

    def write(g, v):
        m = jnp.sum(v) * inv
        ex2 = jnp.sum(v * v) * inv
        var = ex2 - m * m
        o = og * _G + g
        scale = w_ref[o] * lax.rsqrt(var + 1e-5)
        out_ref[0, g] = v * scale + (b_ref[o] - m * scale)

    @pl.when(og < _C1 // _G)
    def _():
        for g in range(_G):
            write(g, x_ref[0, sel1_ref[og * _G + g]])

    @pl.when(og >= _C1 // _G)
    def _():
        for g in range(_G):
            o = og * _G + g
            write(g, x_ref[0, sel1_ref[o]] * x_ref[0, sel2_ref[o]])


@jax.jit
def kernel(x, logits, tau, in_weight, in_bias):
    B, C1, H, W = x.shape
    HW = H * W

    gumbels = -jnp.log(
        jax.random.exponential(jax.random.key(42), (_CAND,), dtype=jnp.float32))
    i_np, j_np = np.triu_indices(_C1, k=1)

    def pad2d(v, fill):
        out = np.full((_RPAD * _CPAD,), fill, dtype=np.float32)
        out[: v.shape[0]] = v
        return out.reshape(_RPAD, _CPAD)

    lp = jnp.zeros((_RPAD * _CPAD,), jnp.float32).at[:_CAND].set(logits)
    lp = lp.reshape(_RPAD, _CPAD)
    gp = jnp.asarray(
        np.full((_RPAD * _CPAD,), _NEG, dtype=np.float32)
    ).at[:_CAND].set(gumbels).reshape(_RPAD, _CPAD)
    ia = jnp.asarray(pad2d(i_np.astype(np.float32), 0.0))
    ja = jnp.asarray(pad2d(j_np.astype(np.float32), 0.0))
    upper = jnp.asarray(np.triu(np.ones((_CPAD, _CPAD), dtype=np.float32)))

    sel = pl.pallas_call(
        _sel_body,
        out_shape=jax.ShapeDtypeStruct((2, 128), jnp.int32),
    )(lp, gp, ia, ja, upper)

    sel1 = jnp.concatenate([jnp.arange(_C1, dtype=jnp.int32), sel[0, :_CE]])
    sel2 = jnp.concatenate([jnp.zeros((_C1,), jnp.int32), sel[1, :_CE]])

    grid_spec = pltpu.PrefetchScalarGridSpec(
        num_scalar_prefetch=4,
        grid=(B, (_C1 + _CE) // _G),
        in_specs=[
            pl.BlockSpec((1, C1, H, W), lambda b, o, s1, s2, w, bi: (b, 0, 0, 0)),
        ],
        out_specs=pl.BlockSpec((1, _G, H, W),
                               lambda b, o, s1, s2, w, bi: (b, o, 0, 0)),
    )
    y = pl.pallas_call(
        functools.partial(_norm_body, hw=HW),
        grid_spec=grid_spec,
        out_shape=jax.ShapeDtypeStruct((B, _C1 + _CE, H, W), jnp.float32),
    )(sel1, sel2, in_weight, in_bias, x)
    return y


# speedup vs baseline: 1.2193x; 1.1002x over previous
import functools
import numpy as np
import jax
import jax.numpy as jnp
from jax import lax
from jax.experimental import pallas as pl
from jax.experimental.pallas import tpu as pltpu

_C1 = 96
_CE = 96
_CAND = _C1 * (_C1 - 1) // 2
_RPAD = 8
_CPAD = 576
_NEG = -1e30
_G = 12


def _sel_body(lp_ref, gp_ref, ia_ref, ja_ref, u_ref, out_ref):
    z = lp_ref[...] + gp_ref[...]
    zmax = jnp.max(z)
    zreal = jnp.where(z < -1e29, zmax, z)
    zmin = jnp.min(zreal)

    def bis(_, carry):
        lo, hi = carry
        mid = 0.5 * (lo + hi)
        cnt = jnp.sum((z >= mid).astype(jnp.float32))
        take = cnt >= float(_CE)
        return (jnp.where(take, mid, lo), jnp.where(take, hi, mid))

    lo, _ = lax.fori_loop(0, 64, bis, (zmin - 1.0, zmax + 1.0))
    mask = (z >= lo).astype(jnp.float32)

    within = jnp.dot(mask, u_ref[...], preferred_element_type=jnp.float32)
    rowsum = jnp.sum(mask, axis=1)
    r = lax.broadcasted_iota(jnp.int32, (_RPAD, _RPAD), 0)
    rp = lax.broadcasted_iota(jnp.int32, (_RPAD, _RPAD), 1)
    offs = jnp.sum(jnp.where(rp < r, rowsum[None, :], 0.0), axis=1)
    ranks = (within + offs[:, None]) * mask
    ranks_i = ranks.astype(jnp.int32)

    e = lax.broadcasted_iota(jnp.int32, (_RPAD, _CPAD, 128), 2) + 1
    onehot = (ranks_i[:, :, None] == e).astype(jnp.float32)
    isel = jnp.sum(jnp.sum(ia_ref[...][:, :, None] * onehot, axis=1), axis=0,
                   keepdims=True)
    jsel = jnp.sum(jnp.sum(ja_ref[...][:, :, None] * onehot, axis=1), axis=0,
                   keepdims=True)
    out_ref[0:1, :] = isel.astype(jnp.int32)
    out_ref[1:2, :] = jsel.astype(jnp.int32)


def _norm_body(sel1_ref, sel2_ref, w_ref, b_ref, x_ref, out_ref, *, hw):
    og = pl.program_id(1)
    inv = 1.0 / float(hw)

    def write(g, v):
        m = jnp.sum(v) * inv
        ex2 = jnp.sum(v * v) * inv
        var = ex2 - m * m
        o = og * _G + g
        scale = w_ref[o] * lax.rsqrt(var + 1e-5)
        out_ref[0, g] = v * scale + (b_ref[o] - m * scale)

    @pl.when(og < _C1 // _G)
    def _():
        for g in range(_G):
            write(g, x_ref[0, sel1_ref[og * _G + g]])

    @pl.when(og >= _C1 // _G)
    def _():
        for g in range(_G):
            o = og * _G + g
            write(g, x_ref[0, sel1_ref[o]] * x_ref[0, sel2_ref[o]])


@jax.jit
def kernel(x, logits, tau, in_weight, in_bias):
    B, C1, H, W = x.shape
    HW = H * W

    gumbels = -jnp.log(
        jax.random.exponential(jax.random.key(42), (_CAND,), dtype=jnp.float32))
    i_np, j_np = np.triu_indices(_C1, k=1)

    def pad2d(v, fill):
        out = np.full((_RPAD * _CPAD,), fill, dtype=np.float32)
        out[: v.shape[0]] = v
        return out.reshape(_RPAD, _CPAD)

    lp = jnp.zeros((_RPAD * _CPAD,), jnp.float32).at[:_CAND].set(logits)
    lp = lp.reshape(_RPAD, _CPAD)
    gp = jnp.asarray(
        np.full((_RPAD * _CPAD,), _NEG, dtype=np.float32)
    ).at[:_CAND].set(gumbels).reshape(_RPAD, _CPAD)
    ia = jnp.asarray(pad2d(i_np.astype(np.float32), 0.0))
    ja = jnp.asarray(pad2d(j_np.astype(np.float32), 0.0))
    upper = jnp.asarray(np.triu(np.ones((_CPAD, _CPAD), dtype=np.float32)))

    sel = pl.pallas_call(
        _sel_body,
        out_shape=jax.ShapeDtypeStruct((2, 128), jnp.int32),
    )(lp, gp, ia, ja, upper)

    sel1 = jnp.concatenate([jnp.arange(_C1, dtype=jnp.int32), sel[0, :_CE]])
    sel2 = jnp.concatenate([jnp.zeros((_C1,), jnp.int32), sel[1, :_CE]])

    grid_spec = pltpu.PrefetchScalarGridSpec(
        num_scalar_prefetch=4,
        grid=(B, (_C1 + _CE) // _G),
        in_specs=[
            pl.BlockSpec((1, C1, H, W), lambda b, o, s1, s2, w, bi: (b, 0, 0, 0)),
        ],
        out_specs=pl.BlockSpec((1, _G, H, W),
                               lambda b, o, s1, s2, w, bi: (b, o, 0, 0)),
    )
    y = pl.pallas_call(
        functools.partial(_norm_body, hw=HW),
        grid_spec=grid_spec,
        out_shape=jax.ShapeDtypeStruct((B, _C1 + _CE, H, W), jnp.float32),
    )(sel1, sel2, in_weight, in_bias, x)
    return y


# speedup vs baseline: 1.2797x; 1.0495x over previous
import functools
import numpy as np
import jax
import jax.numpy as jnp
from jax import lax
from jax.experimental import pallas as pl
from jax.experimental.pallas import tpu as pltpu

_C1 = 96
_CE = 96
_CAND = _C1 * (_C1 - 1) // 2
_RPAD = 8
_CPAD = 576
_NEG = -1e30
_G = 16


def _sel_body(lp_ref, gp_ref, ia_ref, ja_ref, u_ref, out_ref):
    z = lp_ref[...] + gp_ref[...]
    zmax = jnp.max(z)
    zreal = jnp.where(z < -1e29, zmax, z)
    zmin = jnp.min(zreal)

    def bis(_, carry):
        lo, hi = carry
        mid = 0.5 * (lo + hi)
        cnt = jnp.sum((z >= mid).astype(jnp.float32))
        take = cnt >= float(_CE)
        return (jnp.where(take, mid, lo), jnp.where(take, hi, mid))

    lo, _ = lax.fori_loop(0, 64, bis, (zmin - 1.0, zmax + 1.0))
    mask = (z >= lo).astype(jnp.float32)

    within = jnp.dot(mask, u_ref[...], preferred_element_type=jnp.float32)
    rowsum = jnp.sum(mask, axis=1)
    r = lax.broadcasted_iota(jnp.int32, (_RPAD, _RPAD), 0)
    rp = lax.broadcasted_iota(jnp.int32, (_RPAD, _RPAD), 1)
    offs = jnp.sum(jnp.where(rp < r, rowsum[None, :], 0.0), axis=1)
    ranks = (within + offs[:, None]) * mask
    ranks_i = ranks.astype(jnp.int32)

    e = lax.broadcasted_iota(jnp.int32, (_RPAD, _CPAD, 128), 2) + 1
    onehot = (ranks_i[:, :, None] == e).astype(jnp.float32)
    isel = jnp.sum(jnp.sum(ia_ref[...][:, :, None] * onehot, axis=1), axis=0,
                   keepdims=True)
    jsel = jnp.sum(jnp.sum(ja_ref[...][:, :, None] * onehot, axis=1), axis=0,
                   keepdims=True)
    out_ref[0:1, :] = isel.astype(jnp.int32)
    out_ref[1:2, :] = jsel.astype(jnp.int32)


def _norm_body(sel1_ref, sel2_ref, w_ref, b_ref, x_ref, out_ref, *, hw):
    og = pl.program_id(1)
    inv = 1.0 / float(hw)

    def write(g, v):
        m = jnp.sum(v) * inv
        ex2 = jnp.sum(v * v) * inv
        var = ex2 - m * m
        o = og * _G + g
        scale = w_ref[o] * lax.rsqrt(var + 1e-5)
        out_ref[0, g] = v * scale + (b_ref[o] - m * scale)

    @pl.when(og < _C1 // _G)
    def _():
        for g in range(_G):
            write(g, x_ref[0, sel1_ref[og * _G + g]])

    @pl.when(og >= _C1 // _G)
    def _():
        for g in range(_G):
            o = og * _G + g
            write(g, x_ref[0, sel1_ref[o]] * x_ref[0, sel2_ref[o]])


@jax.jit
def kernel(x, logits, tau, in_weight, in_bias):
    B, C1, H, W = x.shape
    HW = H * W

    gumbels = -jnp.log(
        jax.random.exponential(jax.random.key(42), (_CAND,), dtype=jnp.float32))
    i_np, j_np = np.triu_indices(_C1, k=1)

    def pad2d(v, fill):
        out = np.full((_RPAD * _CPAD,), fill, dtype=np.float32)
        out[: v.shape[0]] = v
        return out.reshape(_RPAD, _CPAD)

    lp = jnp.zeros((_RPAD * _CPAD,), jnp.float32).at[:_CAND].set(logits)
    lp = lp.reshape(_RPAD, _CPAD)
    gp = jnp.asarray(
        np.full((_RPAD * _CPAD,), _NEG, dtype=np.float32)
    ).at[:_CAND].set(gumbels).reshape(_RPAD, _CPAD)
    ia = jnp.asarray(pad2d(i_np.astype(np.float32), 0.0))
    ja = jnp.asarray(pad2d(j_np.astype(np.float32), 0.0))
    upper = jnp.asarray(np.triu(np.ones((_CPAD, _CPAD), dtype=np.float32)))

    sel = pl.pallas_call(
        _sel_body,
        out_shape=jax.ShapeDtypeStruct((2, 128), jnp.int32),
    )(lp, gp, ia, ja, upper)

    sel1 = jnp.concatenate([jnp.arange(_C1, dtype=jnp.int32), sel[0, :_CE]])
    sel2 = jnp.concatenate([jnp.zeros((_C1,), jnp.int32), sel[1, :_CE]])

    grid_spec = pltpu.PrefetchScalarGridSpec(
        num_scalar_prefetch=4,
        grid=(B, (_C1 + _CE) // _G),
        in_specs=[
            pl.BlockSpec((1, C1, H, W), lambda b, o, s1, s2, w, bi: (b, 0, 0, 0)),
        ],
        out_specs=pl.BlockSpec((1, _G, H, W),
                               lambda b, o, s1, s2, w, bi: (b, o, 0, 0)),
    )
    y = pl.pallas_call(
        functools.partial(_norm_body, hw=HW),
        grid_spec=grid_spec,
        out_shape=jax.ShapeDtypeStruct((B, _C1 + _CE, H, W), jnp.float32),
    )(sel1, sel2, in_weight, in_bias, x)
    return y
